# Initial kernel scaffold; baseline (speedup 1.0000x reference)
import jax
import jax.numpy as jnp
from jax.experimental import pallas as pl


def kernel(x, edge_index, W1, b1, W2, b2):
    raise NotImplementedError("write your pallas kernel here")


# speedup vs baseline: 5.2590x; 5.2590x over previous
import functools

import jax
import jax.numpy as jnp
from jax import lax
from jax.experimental import pallas as pl
from jax.experimental.pallas import tpu as pltpu

N = 10000
D_FEAT = 128
K = 16
KK = K + 1
GAMMA = 10.0
H1 = 64
H2 = 32

BR = 128
BC = 2048
NPAD = 10240
NC = NPAD // BC
F32_INF = float("inf")
I32_MAX = 0x7FFFFFFF


def _mlp_kernel(x_ref, w1_ref, b1_ref, w2_ref, b2_ref, z_ref):
    h = jnp.maximum(
        lax.dot_general(x_ref[...], w1_ref[...], (((1,), (0,)), ((), ())),
                        preferred_element_type=jnp.float32) + b1_ref[...],
        0.0)
    z_ref[...] = lax.dot_general(h, w2_ref[...], (((1,), (0,)), ((), ())),
                                 preferred_element_type=jnp.float32) + b2_ref[...]


def _topk_kernel(z_ref, zt_ref, oval_ref, oidx_ref, d_ref, rv_ref, ri_ref):
    j = pl.program_id(1)

    @pl.when(j == 0)
    def _init():
        rv_ref[...] = jnp.full((BR, 128), F32_INF, jnp.float32)
        ri_ref[...] = jnp.zeros((BR, 128), jnp.int32)

    z = z_ref[...]
    zt = zt_ref[...]
    sqr = jnp.sum(z * z, axis=1, keepdims=True)
    sqc = jnp.sum(zt * zt, axis=0, keepdims=True)
    dot = lax.dot_general(z, zt, (((1,), (0,)), ((), ())),
                          preferred_element_type=jnp.float32)
    d = sqr + sqc - 2.0 * dot
    colid = j * BC + lax.broadcasted_iota(jnp.int32, (BR, BC), 1)
    d = jnp.where(colid < N, d, F32_INF)
    d_ref[...] = d

    lane = lax.broadcasted_iota(jnp.int32, (BR, 128), 1)
    rv = rv_ref[...]
    ri = ri_ref[...]
    newv = jnp.full((BR, 128), F32_INF, jnp.float32)
    newi = jnp.zeros((BR, 128), jnp.int32)
    mval = jnp.full((BR, 1), -F32_INF, jnp.float32)
    midx = jnp.full((BR, 1), -1, jnp.int32)
    for t in range(KK):
        v = d_ref[...]
        act = (v > mval) | ((v == mval) & (colid > midx))
        vm = jnp.where(act, v, F32_INF)
        actr = (rv > mval) | ((rv == mval) & (ri > midx))
        rvm = jnp.where(actr, rv, F32_INF)
        m = jnp.minimum(jnp.min(vm, axis=1, keepdims=True),
                        jnp.min(rvm, axis=1, keepdims=True))
        i1 = jnp.min(jnp.where(vm == m, colid, I32_MAX), axis=1,
                     keepdims=True)
        i2 = jnp.min(jnp.where(rvm == m, ri, I32_MAX), axis=1,
                     keepdims=True)
        mi = jnp.minimum(i1, i2)
        newv = jnp.where(lane == t, m, newv)
        newi = jnp.where(lane == t, mi, newi)
        mval, midx = m, mi
    rv_ref[...] = newv
    ri_ref[...] = newi

    @pl.when(j == NC - 1)
    def _out():
        oval_ref[...] = newv
        oidx_ref[...] = newi


def kernel(x, edge_index, W1, b1, W2, b2):
    del edge_index
    xpad = jnp.pad(x, ((0, NPAD - N), (0, 0)))

    z = pl.pallas_call(
        _mlp_kernel,
        grid=(NPAD // BR,),
        in_specs=[
            pl.BlockSpec((BR, D_FEAT), lambda i: (i, 0)),
            pl.BlockSpec((D_FEAT, H1), lambda i: (0, 0)),
            pl.BlockSpec((1, H1), lambda i: (0, 0)),
            pl.BlockSpec((H1, H2), lambda i: (0, 0)),
            pl.BlockSpec((1, H2), lambda i: (0, 0)),
        ],
        out_specs=pl.BlockSpec((BR, H2), lambda i: (i, 0)),
        out_shape=jax.ShapeDtypeStruct((NPAD, H2), jnp.float32),
    )(xpad, W1, b1.reshape(1, H1), W2, b2.reshape(1, H2))

    zt = z.T

    vals, idxs = pl.pallas_call(
        _topk_kernel,
        grid=(NPAD // BR, NC),
        in_specs=[
            pl.BlockSpec((BR, H2), lambda i, j: (i, 0)),
            pl.BlockSpec((H2, BC), lambda i, j: (0, j)),
        ],
        out_specs=[
            pl.BlockSpec((BR, 128), lambda i, j: (i, 0)),
            pl.BlockSpec((BR, 128), lambda i, j: (i, 0)),
        ],
        out_shape=[
            jax.ShapeDtypeStruct((NPAD, 128), jnp.float32),
            jax.ShapeDtypeStruct((NPAD, 128), jnp.int32),
        ],
        scratch_shapes=[
            pltpu.VMEM((BR, BC), jnp.float32),
            pltpu.VMEM((BR, 128), jnp.float32),
            pltpu.VMEM((BR, 128), jnp.int32),
        ],
    )(z, zt)

    dst = idxs[:N, 1:KK].reshape(-1)
    d_sel = jnp.maximum(vals[:N, 1:KK], 0.0).reshape(-1)
    src = jnp.repeat(jnp.arange(N, dtype=dst.dtype), K)
    ne_probs = jnp.exp(-GAMMA * d_sel)
    edges = jnp.stack([src, dst])
    row = jnp.minimum(src, dst)
    col = jnp.maximum(src, dst)
    xe = 0.5 * (x[row] + x[col])
    zb = jnp.zeros((N, x.shape[1]), dtype=x.dtype)
    zb = zb.at[row].add(xe).at[col].add(-xe)
    Ldo_xe = zb[row] - zb[col]
    deg = jnp.bincount(row, length=N) + jnp.bincount(col, length=N)
    lup_diag = (deg[row] + deg[col] - 2).astype(x.dtype)
    Lup_xe = lup_diag[:, None] * xe
    return (x, xe, edges, row, col, ne_probs, Ldo_xe, Lup_xe)
